# baseline (speedup 1.0000x reference)
import functools

import jax
import jax.numpy as jnp
from jax import lax
from jax.experimental import pallas as pl
from jax.experimental.pallas import tpu as pltpu
from jax.experimental.pallas import tpu_sc as plsc

_N = 10000
_E = 320000
_H = 128

_NC = 2
_NS = 16
_NW = _NC * _NS
_EPW = _E // _NW
_CHUNK = 128
_BLKC = 8
_NBLK = 10
_NCHUNK = _NBLK * _BLKC
_EPWP = _NCHUNK * _CHUNK
_ROWS_PT = 632
_NPAD = _ROWS_PT * _NS
_TRASH = _N

_BLK = 1000
_GRID = _N // _BLK


def _dense_film_body(x_ref, wf_ref, o_ref):
    h = x_ref[...]
    p = jnp.dot(h, wf_ref[...], preferred_element_type=jnp.float32)
    msg = p[:, :_H]
    gam = p[:, _H:2 * _H]
    bet = p[:, 2 * _H:]
    o_ref[...] = jnp.maximum(gam * msg + bet, 0.0)


def _layernorm(agg, g, b):
    mu = jnp.mean(agg, axis=-1, keepdims=True)
    var = jnp.mean((agg - mu) * (agg - mu), axis=-1, keepdims=True)
    return (agg - mu) * lax.rsqrt(var + 1e-5) * g + b


def _combine_ln_film_body(p_ref, g_ref, b_ref, wf_ref, o_ref):
    pa = p_ref[...]
    h = _layernorm(pa[0] + pa[1], g_ref[...], b_ref[...])
    p = jnp.dot(h, wf_ref[...], preferred_element_type=jnp.float32)
    msg = p[:, :_H]
    gam = p[:, _H:2 * _H]
    bet = p[:, 2 * _H:]
    o_ref[...] = jnp.maximum(gam * msg + bet, 0.0)


def _combine_ln_proj_body(p_ref, g_ref, b_ref, wp_ref, bp_ref, o_ref):
    pa = p_ref[...]
    h = _layernorm(pa[0] + pa[1], g_ref[...], b_ref[...])
    z = jnp.dot(h, wp_ref[...], preferred_element_type=jnp.float32) + bp_ref[...]
    o_ref[...] = jax.nn.sigmoid(z)


_dense_film = pl.pallas_call(
    _dense_film_body,
    grid=(_GRID,),
    in_specs=[
        pl.BlockSpec((_BLK, _H), lambda i: (i, 0)),
        pl.BlockSpec((_H, 3 * _H), lambda i: (0, 0)),
    ],
    out_specs=pl.BlockSpec((_BLK, _H), lambda i: (i, 0)),
    out_shape=jax.ShapeDtypeStruct((_N, _H), jnp.float32),
)

_combine_ln_film = pl.pallas_call(
    _combine_ln_film_body,
    grid=(_GRID,),
    in_specs=[
        pl.BlockSpec((_NC, _BLK, _H), lambda i: (0, i, 0)),
        pl.BlockSpec((1, _H), lambda i: (0, 0)),
        pl.BlockSpec((1, _H), lambda i: (0, 0)),
        pl.BlockSpec((_H, 3 * _H), lambda i: (0, 0)),
    ],
    out_specs=pl.BlockSpec((_BLK, _H), lambda i: (i, 0)),
    out_shape=jax.ShapeDtypeStruct((_N, _H), jnp.float32),
)

_combine_ln_proj = pl.pallas_call(
    _combine_ln_proj_body,
    grid=(_GRID,),
    in_specs=[
        pl.BlockSpec((_NC, _BLK, _H), lambda i: (0, i, 0)),
        pl.BlockSpec((1, _H), lambda i: (0, 0)),
        pl.BlockSpec((1, _H), lambda i: (0, 0)),
        pl.BlockSpec((_H, _H), lambda i: (0, 0)),
        pl.BlockSpec((1, _H), lambda i: (0, 0)),
    ],
    out_specs=pl.BlockSpec((_BLK, _H), lambda i: (i, 0)),
    out_shape=jax.ShapeDtypeStruct((_N, _H), jnp.float32),
)


def _sc_agg_body(m_hbm, src_hbm, dst_hbm, zeros_hbm, out_hbm,
                 srcb0, srcb1, dstb0, dstb1, rows0, rows1, aggs,
                 isem0, isem1, gsem0, gsem1):
    cid = lax.axis_index("c")
    sid = lax.axis_index("s")
    wid = cid * _NS + sid
    stripe = pl.multiple_of(sid * _ROWS_PT, 8)

    srcb = (srcb0, srcb1)
    dstb = (dstb0, dstb1)
    rows = (rows0, rows1)
    isem = (isem0, isem1)
    gsem = (gsem0, gsem1)

    def blk_issue(b, s):
        pltpu.async_copy(src_hbm.at[wid, b], srcb[s], isem[s])
        pltpu.async_copy(dst_hbm.at[wid, b], dstb[s], isem[s])

    def blk_wait(s):
        pltpu.make_async_copy(src_hbm.at[wid, 0], srcb[s], isem[s]).wait()
        pltpu.make_async_copy(dst_hbm.at[wid, 0], dstb[s], isem[s]).wait()

    def gather_issue(s, i, r):
        pltpu.async_copy(m_hbm.at[srcb[s].at[i]], rows[r], gsem[r])

    def gather_wait(s, i, r):
        pltpu.make_async_copy(m_hbm.at[srcb[s].at[i]], rows[r],
                              gsem[r]).wait()

    def scatter(s, i, r):
        pltpu.sync_copy(rows[r], aggs.at[dstb[s].at[i]], add=True)

    pltpu.sync_copy(zeros_hbm, aggs.at[pl.ds(stripe, _ROWS_PT), :])
    blk_issue(0, 0)
    blk_issue(1, 1)
    plsc.subcore_barrier()
    blk_wait(0)
    gather_issue(0, 0, 0)

    def pair_chunks(j, last):
        for t in range(2 * _BLKC):
            s, i = t // _BLKC, t % _BLKC
            r = t % 2
            if t == _BLKC - 1:
                blk_wait(1)
            if t == 2 * _BLKC - 1 and not last:
                blk_wait(0)
            if not (last and t == 2 * _BLKC - 1):
                s2 = ((t + 1) // _BLKC) % 2
                gather_issue(s2, (t + 1) % _BLKC, (t + 1) % 2)
            gather_wait(s, i, r)
            scatter(s, i, r)
            if t == _BLKC - 1 and not last:
                blk_issue(2 * j + 2, 0)
            if t == 2 * _BLKC - 1 and not last:
                blk_issue(2 * j + 3, 1)

    def pipe_body(j, carry):
        pair_chunks(j, False)
        return carry

    lax.fori_loop(0, _NBLK // 2 - 1, pipe_body, 0)
    pair_chunks(_NBLK // 2 - 1, True)
    plsc.subcore_barrier()

    pltpu.sync_copy(aggs.at[pl.ds(stripe, _ROWS_PT), :],
                    out_hbm.at[cid, pl.ds(stripe, _ROWS_PT), :])


_sc_agg = pl.kernel(
    _sc_agg_body,
    out_type=jax.ShapeDtypeStruct((_NC, _NPAD, _H), jnp.float32),
    mesh=plsc.VectorSubcoreMesh(core_axis_name="c", subcore_axis_name="s"),
    scratch_types=[
        pltpu.VMEM((_BLKC, _CHUNK), jnp.int32),
        pltpu.VMEM((_BLKC, _CHUNK), jnp.int32),
        pltpu.VMEM((_BLKC, _CHUNK), jnp.int32),
        pltpu.VMEM((_BLKC, _CHUNK), jnp.int32),
        pltpu.VMEM((_CHUNK, _H), jnp.float32),
        pltpu.VMEM((_CHUNK, _H), jnp.float32),
        pltpu.VMEM_SHARED((_NPAD, _H), jnp.float32),
        pltpu.SemaphoreType.DMA,
        pltpu.SemaphoreType.DMA,
        pltpu.SemaphoreType.DMA,
        pltpu.SemaphoreType.DMA,
    ],
)


def kernel(x, edge_index, W1, F1, g1, b1, W2, F2, g2, b2, Wp, bp):
    pad = ((0, 0), (0, _EPWP - _EPW))
    src = jnp.pad(edge_index[0].reshape(_NW, _EPW), pad)
    src = src.reshape(_NW, _NBLK, _BLKC, _CHUNK)
    dst = jnp.pad(edge_index[1].reshape(_NW, _EPW), pad,
                  constant_values=_TRASH)
    dst = dst.reshape(_NW, _NBLK, _BLKC, _CHUNK)
    zeros = jnp.zeros((_ROWS_PT, _H), jnp.float32)
    wf1 = jnp.concatenate([W1, F1], axis=1)
    wf2 = jnp.concatenate([W2, F2], axis=1)
    g1r = g1.reshape(1, _H)
    b1r = b1.reshape(1, _H)
    g2r = g2.reshape(1, _H)
    b2r = b2.reshape(1, _H)
    bpr = bp.reshape(1, _H)

    m1 = _dense_film(x, wf1)
    p1 = _sc_agg(m1, src, dst, zeros)
    m2 = _combine_ln_film(p1, g1r, b1r, wf2)
    p2 = _sc_agg(m2, src, dst, zeros)
    out = _combine_ln_proj(p2, g2r, b2r, Wp, bpr)
    return out


# speedup vs baseline: 2.8124x; 2.8124x over previous
import functools

import jax
import jax.numpy as jnp
from jax import lax
from jax.experimental import pallas as pl
from jax.experimental.pallas import tpu as pltpu
from jax.experimental.pallas import tpu_sc as plsc

_N = 10000
_E = 320000
_H = 128

_NC = 2
_NS = 16
_NW = _NC * _NS
_EPW = _E // _NW
_CHUNK = 80
_NCHUNK = _EPW // _CHUNK
_ROWS_PT = 632
_NPAD = _ROWS_PT * _NS

_BLK = 1000
_GRID = _N // _BLK


def _film(h, w_ref, f_ref):
    msg = jnp.dot(h, w_ref[...], preferred_element_type=jnp.float32)
    film = jnp.dot(h, f_ref[...], preferred_element_type=jnp.float32)
    gam = film[:, :_H]
    bet = film[:, _H:]
    return jnp.maximum(gam * msg + bet, 0.0)


def _dense_film_body(x_ref, w_ref, f_ref, o_ref):
    o_ref[...] = _film(x_ref[...], w_ref, f_ref)


def _layernorm(agg, g, b):
    mu = jnp.mean(agg, axis=-1, keepdims=True)
    var = jnp.mean((agg - mu) * (agg - mu), axis=-1, keepdims=True)
    return (agg - mu) * lax.rsqrt(var + 1e-5) * g + b


def _combine_ln_film_body(p_ref, g_ref, b_ref, w_ref, f_ref, o_ref):
    pa = p_ref[...]
    h = _layernorm(pa[0] + pa[1], g_ref[...], b_ref[...])
    o_ref[...] = _film(h, w_ref, f_ref)


def _combine_ln_proj_body(p_ref, g_ref, b_ref, wp_ref, bp_ref, o_ref):
    pa = p_ref[...]
    h = _layernorm(pa[0] + pa[1], g_ref[...], b_ref[...])
    z = jnp.dot(h, wp_ref[...], preferred_element_type=jnp.float32) + bp_ref[...]
    o_ref[...] = jax.nn.sigmoid(z)


_dense_film = pl.pallas_call(
    _dense_film_body,
    grid=(_GRID,),
    in_specs=[
        pl.BlockSpec((_BLK, _H), lambda i: (i, 0)),
        pl.BlockSpec((_H, _H), lambda i: (0, 0)),
        pl.BlockSpec((_H, 2 * _H), lambda i: (0, 0)),
    ],
    out_specs=pl.BlockSpec((_BLK, _H), lambda i: (i, 0)),
    out_shape=jax.ShapeDtypeStruct((_N, _H), jnp.float32),
)

_combine_ln_film = pl.pallas_call(
    _combine_ln_film_body,
    grid=(_GRID,),
    in_specs=[
        pl.BlockSpec((_NC, _BLK, _H), lambda i: (0, i, 0)),
        pl.BlockSpec((1, _H), lambda i: (0, 0)),
        pl.BlockSpec((1, _H), lambda i: (0, 0)),
        pl.BlockSpec((_H, _H), lambda i: (0, 0)),
        pl.BlockSpec((_H, 2 * _H), lambda i: (0, 0)),
    ],
    out_specs=pl.BlockSpec((_BLK, _H), lambda i: (i, 0)),
    out_shape=jax.ShapeDtypeStruct((_N, _H), jnp.float32),
)

_combine_ln_proj = pl.pallas_call(
    _combine_ln_proj_body,
    grid=(_GRID,),
    in_specs=[
        pl.BlockSpec((_NC, _BLK, _H), lambda i: (0, i, 0)),
        pl.BlockSpec((1, _H), lambda i: (0, 0)),
        pl.BlockSpec((1, _H), lambda i: (0, 0)),
        pl.BlockSpec((_H, _H), lambda i: (0, 0)),
        pl.BlockSpec((1, _H), lambda i: (0, 0)),
    ],
    out_specs=pl.BlockSpec((_BLK, _H), lambda i: (i, 0)),
    out_shape=jax.ShapeDtypeStruct((_N, _H), jnp.float32),
)


def _sc_agg_body(m_hbm, src_hbm, dst_hbm, zeros_hbm, out_hbm,
                 srcall, dstall, rows0, rows1, aggs, gsem0, gsem1):
    cid = lax.axis_index("c")
    sid = lax.axis_index("s")
    wid = cid * _NS + sid
    stripe = pl.multiple_of(sid * _ROWS_PT, 8)

    pltpu.sync_copy(src_hbm.at[wid], srcall)
    pltpu.sync_copy(dst_hbm.at[wid], dstall)

    pltpu.sync_copy(zeros_hbm, aggs.at[pl.ds(stripe, _ROWS_PT), :])
    plsc.subcore_barrier()

    def gather_issue(c, rows, gsem):
        off = pl.multiple_of(c * _CHUNK, 8)
        pltpu.async_copy(m_hbm.at[srcall.at[pl.ds(off, _CHUNK)]], rows, gsem)

    def gather_wait(c, rows, gsem):
        off = pl.multiple_of(c * _CHUNK, 8)
        pltpu.make_async_copy(m_hbm.at[srcall.at[pl.ds(off, _CHUNK)]],
                              rows, gsem).wait()


    def scatter(c, rows):
        pltpu.sync_copy(rows, aggs.at[dstall.at[c]], add=True)

    gather_issue(0, rows0, gsem0)

    def pipe_body(k, carry):
        a = 2 * k
        gather_issue(a + 1, rows1, gsem1)
        gather_wait(a, rows0, gsem0)
        scatter(a, rows0)
        gather_issue(a + 2, rows0, gsem0)
        gather_wait(a + 1, rows1, gsem1)
        scatter(a + 1, rows1)
        return carry

    lax.fori_loop(0, (_NCHUNK - 1) // 2, pipe_body, 0)
    gather_wait(_NCHUNK - 1, rows0, gsem0)
    scatter(_NCHUNK - 1, rows0)
    plsc.subcore_barrier()

    pltpu.sync_copy(aggs.at[pl.ds(stripe, _ROWS_PT), :],
                    out_hbm.at[cid, pl.ds(stripe, _ROWS_PT), :])


_sc_agg = pl.kernel(
    _sc_agg_body,
    out_type=jax.ShapeDtypeStruct((_NC, _NPAD, _H), jnp.float32),
    mesh=plsc.VectorSubcoreMesh(core_axis_name="c", subcore_axis_name="s"),
    scratch_types=[
        pltpu.VMEM((_EPW,), jnp.int32),
        pltpu.VMEM((_NCHUNK, _CHUNK), jnp.int32),
        pltpu.VMEM((_CHUNK, _H), jnp.float32),
        pltpu.VMEM((_CHUNK, _H), jnp.float32),
        pltpu.VMEM_SHARED((_NPAD, _H), jnp.float32),
        pltpu.SemaphoreType.DMA,
        pltpu.SemaphoreType.DMA,
    ],
)


def kernel(x, edge_index, W1, F1, g1, b1, W2, F2, g2, b2, Wp, bp):
    src = edge_index[0].reshape(_NW, _EPW)
    dst = edge_index[1].reshape(_NW, _NCHUNK, _CHUNK)
    zeros = jnp.zeros((_ROWS_PT, _H), jnp.float32)
    g1r = g1.reshape(1, _H)
    b1r = b1.reshape(1, _H)
    g2r = g2.reshape(1, _H)
    b2r = b2.reshape(1, _H)
    bpr = bp.reshape(1, _H)

    m1 = _dense_film(x, W1, F1)
    p1 = _sc_agg(m1, src, dst, zeros)
    m2 = _combine_ln_film(p1, g1r, b1r, W2, F2)
    p2 = _sc_agg(m2, src, dst, zeros)
    out = _combine_ln_proj(p2, g2r, b2r, Wp, bpr)
    return out
